# baseline (speedup 1.0000x reference)
import jax
import jax.numpy as jnp
from jax import lax
from jax.experimental import pallas as pl
from jax.experimental.pallas import tpu as pltpu
from jax.experimental.pallas import tpu_sc as plsc

VOCAB = 1000000
D = 64
B = 16384
HIST = 50
PAD = 64
NW = 32
BAGS_PER_W = B // NW
GROUP_BAGS = 2
GROUP_IDX = GROUP_BAGS * PAD
GROUPS = BAGS_PER_W // GROUP_BAGS
LANES = 16
DV = D // LANES


def _lane_total(x, scratch):
    lanes = lax.iota(jnp.int32, LANES)
    for s in (8, 4, 2, 1):
        scratch[pl.ds(0, LANES)] = x
        x = x + plsc.load_gather(scratch, [lanes ^ s])
    return x


def _worker_body(table, idx_hbm, ctx_hbm, out_hbm,
                 idx_v, ctx_v, row0_v, rows_v, out_v, red_v, sem0, sem1):
    nc = 2
    wid = lax.axis_index("s") * nc + lax.axis_index("c")

    pltpu.sync_copy(idx_hbm.at[wid], idx_v)
    pltpu.sync_copy(ctx_hbm.at[wid], ctx_v)
    pltpu.sync_copy(table.at[pl.ds(0, 1)], row0_v)

    def gather(g, buf, sem):
        return pltpu.async_copy(table.at[idx_v.at[g]], rows_v.at[buf], sem)

    def wait(buf, sem):
        pltpu.make_async_copy(table.at[idx_v.at[0]], rows_v.at[buf], sem).wait()

    r0 = [row0_v[0, pl.ds(j * LANES, LANES)] for j in range(DV)]
    zero = jnp.zeros((LANES,), jnp.float32)
    UNROLL = 4

    def process_group(g, buf):
        rows = rows_v.at[buf]

        def bag_body(bb, carry):
            b_local = g * GROUP_BAGS + bb
            base = bb * PAD

            def ent_body(e4, accs):
                a = list(accs)
                for u in range(UNROLL):
                    e = base + e4 * UNROLL + u
                    for j in range(DV):
                        a[j] = a[j] + rows[e, pl.ds(j * LANES, LANES)]
                return tuple(a)

            accs = lax.fori_loop(0, PAD // UNROLL, ent_body,
                                 (zero, zero, zero, zero))
            zc = jnp.zeros((LANES,), jnp.int32)
            for t in range(PAD // LANES):
                iv = idx_v[g, pl.ds(base + t * LANES, LANES)]
                zc = zc + jnp.where(iv == 0, 1, 0).astype(jnp.int32)
            zf = _lane_total(zc, red_v).astype(jnp.float32)
            cv = plsc.load_gather(
                ctx_v, [jnp.full((LANES,), b_local, jnp.int32)]
            ).astype(jnp.float32)
            inv = 1.0 / cv
            for j in range(DV):
                out_v[b_local, pl.ds(j * LANES, LANES)] = (accs[j] - zf * r0[j]) * inv
            return carry

        lax.fori_loop(0, GROUP_BAGS, bag_body, 0)

    gather(0, 0, sem0)

    def step(k, carry):
        gather(2 * k + 1, 1, sem1)
        wait(0, sem0)
        process_group(2 * k, 0)

        @pl.when(k < GROUPS // 2 - 1)
        def _():
            gather(2 * k + 2, 0, sem0)

        wait(1, sem1)
        process_group(2 * k + 1, 1)
        return carry

    lax.fori_loop(0, GROUPS // 2, step, 0)

    pltpu.sync_copy(out_v, out_hbm.at[wid])


@jax.jit
def _cbow_sc(table, idx, ctx):
    mesh = plsc.VectorSubcoreMesh(core_axis_name="c", subcore_axis_name="s")
    f = pl.kernel(
        _worker_body,
        out_type=jax.ShapeDtypeStruct((NW, BAGS_PER_W, D), jnp.float32),
        mesh=mesh,
        scratch_types=[
            pltpu.VMEM((GROUPS, GROUP_IDX), jnp.int32),
            pltpu.VMEM((BAGS_PER_W,), jnp.int32),
            pltpu.VMEM((1, D), jnp.float32),
            pltpu.VMEM((2, GROUP_IDX, D), jnp.float32),
            pltpu.VMEM((BAGS_PER_W, D), jnp.float32),
            pltpu.VMEM((LANES,), jnp.int32),
            pltpu.SemaphoreType.DMA,
            pltpu.SemaphoreType.DMA,
        ],
        compiler_params=pltpu.CompilerParams(
            needs_layout_passes=False, use_tc_tiling_on_sc=False
        ),
    )
    return f(table, idx, ctx)


def kernel(embedding, bow, context_size):
    bow = bow.astype(jnp.int32)
    bow = jnp.pad(bow, ((0, 0), (0, PAD - HIST)))
    idx = bow.reshape(NW, GROUPS, GROUP_IDX)
    ctx = context_size.astype(jnp.int32).reshape(NW, BAGS_PER_W)
    out = _cbow_sc(embedding, idx, ctx)
    return out.reshape(B, D)


# speedup vs baseline: 1.0016x; 1.0016x over previous
import jax
import jax.numpy as jnp
from jax import lax
from jax.experimental import pallas as pl
from jax.experimental.pallas import tpu as pltpu
from jax.experimental.pallas import tpu_sc as plsc

VOCAB = 1000000
D = 64
B = 16384
HIST = 50
PAD = 64
NW = 32
BAGS_PER_W = B // NW
GROUP_BAGS = 2
GROUP_IDX = GROUP_BAGS * PAD
GROUPS = BAGS_PER_W // GROUP_BAGS
LANES = 16
DV = D // LANES
NBUF = 4


def _lane_total(x, scratch):
    lanes = lax.iota(jnp.int32, LANES)
    for s in (8, 4, 2, 1):
        scratch[pl.ds(0, LANES)] = x
        x = x + plsc.load_gather(scratch, [lanes ^ s])
    return x


def _worker_body(table, idx_hbm, ctx_hbm, out_hbm,
                 idx_v, ctx_v, row0_v, rows_v, out_v, red_v,
                 sem0, sem1, sem2, sem3):
    nc = 2
    wid = lax.axis_index("s") * nc + lax.axis_index("c")

    pltpu.sync_copy(idx_hbm.at[wid], idx_v)
    pltpu.sync_copy(ctx_hbm.at[wid], ctx_v)
    pltpu.sync_copy(table.at[pl.ds(0, 1)], row0_v)

    def gather(g, buf, sem):
        return pltpu.async_copy(table.at[idx_v.at[g]], rows_v.at[buf], sem)

    def wait(buf, sem):
        pltpu.make_async_copy(table.at[idx_v.at[0]], rows_v.at[buf], sem).wait()

    r0 = [row0_v[0, pl.ds(j * LANES, LANES)] for j in range(DV)]
    zero = jnp.zeros((LANES,), jnp.float32)
    UNROLL = 4

    def process_group(g, buf):
        rows = rows_v.at[buf]

        def bag_body(bb, carry):
            b_local = g * GROUP_BAGS + bb
            base = bb * PAD

            def ent_body(e4, accs):
                a = list(accs)
                for u in range(UNROLL):
                    e = base + e4 * UNROLL + u
                    for j in range(DV):
                        a[j] = a[j] + rows[e, pl.ds(j * LANES, LANES)]
                return tuple(a)

            accs = lax.fori_loop(0, PAD // UNROLL, ent_body,
                                 (zero, zero, zero, zero))
            zc = jnp.zeros((LANES,), jnp.int32)
            for t in range(PAD // LANES):
                iv = idx_v[g, pl.ds(base + t * LANES, LANES)]
                zc = zc + jnp.where(iv == 0, 1, 0).astype(jnp.int32)
            zf = _lane_total(zc, red_v).astype(jnp.float32)
            cv = plsc.load_gather(
                ctx_v, [jnp.full((LANES,), b_local, jnp.int32)]
            ).astype(jnp.float32)
            inv = 1.0 / cv
            for j in range(DV):
                out_v[b_local, pl.ds(j * LANES, LANES)] = (accs[j] - zf * r0[j]) * inv
            return carry

        lax.fori_loop(0, GROUP_BAGS, bag_body, 0)

    sems = (sem0, sem1, sem2, sem3)
    for b in range(NBUF - 1):
        gather(b, b, sems[b])

    def step(k, carry):
        for u in range(NBUF):
            g = NBUF * k + u

            @pl.when(g + NBUF - 1 < GROUPS)
            def _():
                gather(g + NBUF - 1, (u + NBUF - 1) % NBUF,
                       sems[(u + NBUF - 1) % NBUF])

            wait(u, sems[u])
            process_group(g, u)
        return carry

    lax.fori_loop(0, GROUPS // NBUF, step, 0)

    pltpu.sync_copy(out_v, out_hbm.at[wid])


@jax.jit
def _cbow_sc(table, idx, ctx):
    mesh = plsc.VectorSubcoreMesh(core_axis_name="c", subcore_axis_name="s")
    f = pl.kernel(
        _worker_body,
        out_type=jax.ShapeDtypeStruct((NW, BAGS_PER_W, D), jnp.float32),
        mesh=mesh,
        scratch_types=[
            pltpu.VMEM((GROUPS, GROUP_IDX), jnp.int32),
            pltpu.VMEM((BAGS_PER_W,), jnp.int32),
            pltpu.VMEM((1, D), jnp.float32),
            pltpu.VMEM((NBUF, GROUP_IDX, D), jnp.float32),
            pltpu.VMEM((BAGS_PER_W, D), jnp.float32),
            pltpu.VMEM((LANES,), jnp.int32),
            pltpu.SemaphoreType.DMA,
            pltpu.SemaphoreType.DMA,
            pltpu.SemaphoreType.DMA,
            pltpu.SemaphoreType.DMA,
        ],
        compiler_params=pltpu.CompilerParams(
            needs_layout_passes=False, use_tc_tiling_on_sc=False
        ),
    )
    return f(table, idx, ctx)


def kernel(embedding, bow, context_size):
    bow = bow.astype(jnp.int32)
    bow = jnp.pad(bow, ((0, 0), (0, PAD - HIST)))
    idx = bow.reshape(NW, GROUPS, GROUP_IDX)
    ctx = context_size.astype(jnp.int32).reshape(NW, BAGS_PER_W)
    out = _cbow_sc(embedding, idx, ctx)
    return out.reshape(B, D)


# speedup vs baseline: 6.9852x; 6.9744x over previous
import jax
import jax.numpy as jnp
from jax import lax
from jax.experimental import pallas as pl
from jax.experimental.pallas import tpu as pltpu
from jax.experimental.pallas import tpu_sc as plsc

VOCAB = 1000000
D = 64
B = 16384
HIST = 50
NW = 32
BAGS_PER_W = B // NW
GROUP_BAGS = 8
GROUP_IDX = GROUP_BAGS * HIST
GROUPS = BAGS_PER_W // GROUP_BAGS
LANES = 16
DV = D // LANES
NBUF = 2
UNROLL = 5


def _lane_total(x, scratch):
    lanes = lax.iota(jnp.int32, LANES)
    for s in (8, 4, 2, 1):
        scratch[pl.ds(0, LANES)] = x
        x = x + plsc.load_gather(scratch, [lanes ^ s])
    return x


def _worker_body(table, idx_hbm, ctx_hbm, out_hbm,
                 idx_v, ctx_v, row0_v, rows_v, out_v, red_v, sem0, sem1):
    nc = 2
    wid = lax.axis_index("s") * nc + lax.axis_index("c")

    pltpu.sync_copy(idx_hbm.at[wid], idx_v)
    pltpu.sync_copy(ctx_hbm.at[wid], ctx_v)
    pltpu.sync_copy(table.at[pl.ds(0, 1)], row0_v)

    def gather(g, buf, sem):
        return pltpu.async_copy(table.at[idx_v.at[g]], rows_v.at[buf], sem)

    def wait(buf, sem):
        pltpu.make_async_copy(table.at[idx_v.at[0]], rows_v.at[buf], sem).wait()

    r0 = [row0_v[0, pl.ds(j * LANES, LANES)] for j in range(DV)]
    zero = jnp.zeros((LANES,), jnp.float32)
    lanes = lax.iota(jnp.int32, LANES)

    def process_group(g, buf):
        rows = rows_v.at[buf]

        def bag_body(bb, carry):
            b_local = g * GROUP_BAGS + bb
            base = bb * HIST

            def ent_body(e5, accs):
                a = list(accs)
                for u in range(UNROLL):
                    e = base + e5 * UNROLL + u
                    for j in range(DV):
                        a[j] = a[j] + rows[e, pl.ds(j * LANES, LANES)]
                return tuple(a)

            accs = lax.fori_loop(0, HIST // UNROLL, ent_body,
                                 (zero, zero, zero, zero))
            one = jnp.ones((LANES,), jnp.int32)
            nil = jnp.zeros((LANES,), jnp.int32)
            zc = nil
            for t in range(3):
                iv = idx_v[g, pl.ds(base + t * LANES, LANES)]
                zc = zc + jnp.where(iv == 0, one, nil)
            iv = idx_v[g, pl.ds(base + HIST - LANES, LANES)]
            zc = zc + jnp.where((iv == 0) & (lanes >= 4 * LANES - HIST),
                                one, nil)
            zf = _lane_total(zc, red_v).astype(jnp.float32)
            cv = plsc.load_gather(
                ctx_v, [jnp.full((LANES,), b_local, jnp.int32)]
            ).astype(jnp.float32)
            inv = 1.0 / cv
            for j in range(DV):
                out_v[b_local, pl.ds(j * LANES, LANES)] = (accs[j] - zf * r0[j]) * inv
            return carry

        lax.fori_loop(0, GROUP_BAGS, bag_body, 0)

    sems = (sem0, sem1)
    for b in range(NBUF - 1):
        gather(b, b, sems[b])

    def step(k, carry):
        for u in range(NBUF):
            g = NBUF * k + u

            @pl.when(g + NBUF - 1 < GROUPS)
            def _():
                gather(g + NBUF - 1, (u + NBUF - 1) % NBUF,
                       sems[(u + NBUF - 1) % NBUF])

            wait(u, sems[u])
            process_group(g, u)
        return carry

    lax.fori_loop(0, GROUPS // NBUF, step, 0)

    pltpu.sync_copy(out_v, out_hbm.at[wid])


@jax.jit
def _cbow_sc(table, idx, ctx):
    mesh = plsc.VectorSubcoreMesh(core_axis_name="c", subcore_axis_name="s")
    f = pl.kernel(
        _worker_body,
        out_type=jax.ShapeDtypeStruct((NW, BAGS_PER_W, D), jnp.float32),
        mesh=mesh,
        scratch_types=[
            pltpu.VMEM((GROUPS, GROUP_IDX), jnp.int32),
            pltpu.VMEM((BAGS_PER_W,), jnp.int32),
            pltpu.VMEM((1, D), jnp.float32),
            pltpu.VMEM((NBUF, GROUP_IDX, D), jnp.float32),
            pltpu.VMEM((BAGS_PER_W, D), jnp.float32),
            pltpu.VMEM((LANES,), jnp.int32),
            pltpu.SemaphoreType.DMA,
            pltpu.SemaphoreType.DMA,
        ],
        compiler_params=pltpu.CompilerParams(
            needs_layout_passes=False, use_tc_tiling_on_sc=False
        ),
    )
    return f(table, idx, ctx)


def kernel(embedding, bow, context_size):
    idx = bow.astype(jnp.int32).reshape(NW, GROUPS, GROUP_IDX)
    ctx = context_size.astype(jnp.int32).reshape(NW, BAGS_PER_W)
    out = _cbow_sc(embedding, idx, ctx)
    return out.reshape(B, D)


# speedup vs baseline: 6.9964x; 1.0016x over previous
import jax
import jax.numpy as jnp
from jax import lax
from jax.experimental import pallas as pl
from jax.experimental.pallas import tpu as pltpu
from jax.experimental.pallas import tpu_sc as plsc

VOCAB = 1000000
D = 64
B = 16384
HIST = 50
NW = 32
BAGS_PER_W = B // NW
GROUP_BAGS = 8
GROUP_IDX = GROUP_BAGS * HIST
GROUPS = BAGS_PER_W // GROUP_BAGS
LANES = 16
DV = D // LANES
NBUF = 2
IDX_PER_W = BAGS_PER_W * HIST
UNROLL = 5


def _lane_total(x, scratch):
    lanes = lax.iota(jnp.int32, LANES)
    for s in (8, 4, 2, 1):
        scratch[pl.ds(0, LANES)] = x
        x = x + plsc.load_gather(scratch, [lanes ^ s])
    return x


def _worker_body(table, idx_hbm, ctx_hbm, out_hbm,
                 idx_v, ctx_v, row0_v, rows_v, out_v, red_v, sem0, sem1):
    nc = 2
    wid = lax.axis_index("s") * nc + lax.axis_index("c")

    pltpu.sync_copy(idx_hbm.at[pl.ds(wid * IDX_PER_W, IDX_PER_W)], idx_v)
    pltpu.sync_copy(ctx_hbm.at[pl.ds(wid * BAGS_PER_W, BAGS_PER_W)], ctx_v)
    pltpu.sync_copy(table.at[pl.ds(0, 1)], row0_v)

    def gather(g, buf, sem):
        return pltpu.async_copy(
            table.at[idx_v.at[pl.ds(g * GROUP_IDX, GROUP_IDX)]],
            rows_v.at[buf], sem)

    def wait(buf, sem):
        pltpu.make_async_copy(
            table.at[idx_v.at[pl.ds(0, GROUP_IDX)]],
            rows_v.at[buf], sem).wait()

    r0 = [row0_v[0, pl.ds(j * LANES, LANES)] for j in range(DV)]
    zero = jnp.zeros((LANES,), jnp.float32)
    lanes = lax.iota(jnp.int32, LANES)

    def process_group(g, buf):
        rows = rows_v.at[buf]

        def bag_body(bb, carry):
            b_local = g * GROUP_BAGS + bb
            base = bb * HIST

            def ent_body(e5, accs):
                a = list(accs)
                for u in range(UNROLL):
                    e = base + e5 * UNROLL + u
                    for j in range(DV):
                        a[j] = a[j] + rows[e, pl.ds(j * LANES, LANES)]
                return tuple(a)

            accs = lax.fori_loop(0, HIST // UNROLL, ent_body,
                                 (zero, zero, zero, zero))
            one = jnp.ones((LANES,), jnp.int32)
            nil = jnp.zeros((LANES,), jnp.int32)
            zc = nil
            for t in range(3):
                iv = idx_v[pl.ds(g * GROUP_IDX + base + t * LANES, LANES)]
                zc = zc + jnp.where(iv == 0, one, nil)
            iv = idx_v[pl.ds(g * GROUP_IDX + base + HIST - LANES, LANES)]
            zc = zc + jnp.where((iv == 0) & (lanes >= 4 * LANES - HIST),
                                one, nil)
            zf = _lane_total(zc, red_v).astype(jnp.float32)
            cv = plsc.load_gather(
                ctx_v, [jnp.full((LANES,), b_local, jnp.int32)]
            ).astype(jnp.float32)
            inv = 1.0 / cv
            for j in range(DV):
                out_v[b_local, pl.ds(j * LANES, LANES)] = (accs[j] - zf * r0[j]) * inv
            return carry

        lax.fori_loop(0, GROUP_BAGS, bag_body, 0)

    sems = (sem0, sem1)
    for b in range(NBUF - 1):
        gather(b, b, sems[b])

    def step(k, carry):
        for u in range(NBUF):
            g = NBUF * k + u

            @pl.when(g + NBUF - 1 < GROUPS)
            def _():
                gather(g + NBUF - 1, (u + NBUF - 1) % NBUF,
                       sems[(u + NBUF - 1) % NBUF])

            wait(u, sems[u])
            process_group(g, u)
        return carry

    lax.fori_loop(0, GROUPS // NBUF, step, 0)

    pltpu.sync_copy(out_v, out_hbm.at[pl.ds(wid * BAGS_PER_W, BAGS_PER_W)])


@jax.jit
def _cbow_sc(table, idx, ctx):
    mesh = plsc.VectorSubcoreMesh(core_axis_name="c", subcore_axis_name="s")
    f = pl.kernel(
        _worker_body,
        out_type=jax.ShapeDtypeStruct((B, D), jnp.float32),
        mesh=mesh,
        scratch_types=[
            pltpu.VMEM((IDX_PER_W,), jnp.int32),
            pltpu.VMEM((BAGS_PER_W,), jnp.int32),
            pltpu.VMEM((1, D), jnp.float32),
            pltpu.VMEM((NBUF, GROUP_IDX, D), jnp.float32),
            pltpu.VMEM((BAGS_PER_W, D), jnp.float32),
            pltpu.VMEM((LANES,), jnp.int32),
            pltpu.SemaphoreType.DMA,
            pltpu.SemaphoreType.DMA,
        ],
        compiler_params=pltpu.CompilerParams(
            needs_layout_passes=False, use_tc_tiling_on_sc=False
        ),
    )
    return f(table, idx, ctx)


def kernel(embedding, bow, context_size):
    idx = bow.astype(jnp.int32).reshape(B * HIST)
    ctx = context_size.astype(jnp.int32)
    return _cbow_sc(embedding, idx, ctx)
